# Initial kernel scaffold; baseline (speedup 1.0000x reference)
import jax
import jax.numpy as jnp
from jax.experimental import pallas as pl


def kernel(x, W_base, b_base, router_W, router_b, lora_A, lora_B):
    raise NotImplementedError("write your pallas kernel here")


# speedup vs baseline: 4.4308x; 4.4308x over previous
import functools

import jax
import jax.numpy as jnp
import numpy as np
from jax.experimental import pallas as pl

IN_FEATURES = 2048
OUT_FEATURES = 2048
NUM_EXPERTS = 8
LORA_RANK = 16
SPARSEGEN_LAMBDA = 0.5
LORA_SCALING = 2.0

LANES = 128
BT = 512


def _body(x_ref, wt_ref, b_ref, rw_ref, rb_ref, acat_ref, bcat_ref, rep_ref,
          o_ref):
    x = x_ref[...]

    z = jnp.dot(x, rw_ref[...], preferred_element_type=jnp.float32)
    z = (z + rb_ref[...]) * (1.0 / (1.0 - SPARSEGEN_LAMBDA))
    lane = jax.lax.broadcasted_iota(jnp.int32, z.shape, dimension=1)
    mask = lane < NUM_EXPERTS
    active = mask.astype(jnp.float32)
    tau = jnp.zeros((z.shape[0], 1), jnp.float32)
    for _ in range(NUM_EXPERTS):
        n = jnp.sum(active, axis=-1, keepdims=True)
        s = jnp.sum(z * active, axis=-1, keepdims=True)
        tau = (s - 1.0) / n
        active = jnp.where(mask & (z > tau), 1.0, 0.0)
    w = jnp.where(mask, jnp.maximum(z - tau, 0.0), 0.0)

    wrep = jnp.dot(w, rep_ref[...], preferred_element_type=jnp.float32)
    h = jnp.dot(x, acat_ref[...], preferred_element_type=jnp.float32)
    eo = jnp.dot(wrep * h, bcat_ref[...], preferred_element_type=jnp.float32)

    base = jnp.dot(x, wt_ref[...], preferred_element_type=jnp.float32)
    o_ref[...] = base + b_ref[...] + LORA_SCALING * eo


@functools.partial(jax.jit, static_argnames=())
def kernel(x, W_base, b_base, router_W, router_b, lora_A, lora_B):
    T = x.shape[0]
    Wt = W_base.T
    b2 = b_base.reshape(1, OUT_FEATURES)
    rw = jnp.zeros((IN_FEATURES, LANES), jnp.float32).at[:, :NUM_EXPERTS].set(
        router_W)
    rb = jnp.zeros((1, LANES), jnp.float32).at[0, :NUM_EXPERTS].set(router_b)
    acat = lora_A.transpose(1, 0, 2).reshape(IN_FEATURES,
                                             NUM_EXPERTS * LORA_RANK)
    bcat = lora_B.reshape(NUM_EXPERTS * LORA_RANK, OUT_FEATURES)
    rep = np.zeros((LANES, LANES), np.float32)
    for e in range(NUM_EXPERTS):
        rep[e, e * LORA_RANK:(e + 1) * LORA_RANK] = 1.0
    rep = jnp.asarray(rep)

    grid = (T // BT,)
    out = pl.pallas_call(
        _body,
        grid=grid,
        in_specs=[
            pl.BlockSpec((BT, IN_FEATURES), lambda i: (i, 0)),
            pl.BlockSpec((IN_FEATURES, OUT_FEATURES), lambda i: (0, 0)),
            pl.BlockSpec((1, OUT_FEATURES), lambda i: (0, 0)),
            pl.BlockSpec((IN_FEATURES, LANES), lambda i: (0, 0)),
            pl.BlockSpec((1, LANES), lambda i: (0, 0)),
            pl.BlockSpec((IN_FEATURES, NUM_EXPERTS * LORA_RANK),
                         lambda i: (0, 0)),
            pl.BlockSpec((NUM_EXPERTS * LORA_RANK, OUT_FEATURES),
                         lambda i: (0, 0)),
            pl.BlockSpec((LANES, LANES), lambda i: (0, 0)),
        ],
        out_specs=pl.BlockSpec((BT, OUT_FEATURES), lambda i: (i, 0)),
        out_shape=jax.ShapeDtypeStruct((T, OUT_FEATURES), jnp.float32),
    )(x, Wt, b2, rw, rb, acat, bcat, rep)
    return out


# speedup vs baseline: 4.7347x; 1.0686x over previous
import functools

import jax
import jax.numpy as jnp
import numpy as np
from jax.experimental import pallas as pl

IN_FEATURES = 2048
OUT_FEATURES = 2048
NUM_EXPERTS = 8
LORA_RANK = 16
SPARSEGEN_LAMBDA = 0.5
LORA_SCALING = 2.0

LANES = 128
BT = 512


def _body(x_ref, wt_ref, b_ref, rw_ref, rb_ref, acat_ref, bcat_ref, rep_ref,
          o_ref):
    x = x_ref[...]
    xb = x.astype(jnp.bfloat16)

    z = jnp.dot(x, rw_ref[...], preferred_element_type=jnp.float32)
    z = (z + rb_ref[...]) * (1.0 / (1.0 - SPARSEGEN_LAMBDA))
    lane = jax.lax.broadcasted_iota(jnp.int32, z.shape, dimension=1)
    mask = lane < NUM_EXPERTS
    active = mask.astype(jnp.float32)
    tau = jnp.zeros((z.shape[0], 1), jnp.float32)
    for _ in range(NUM_EXPERTS):
        n = jnp.sum(active, axis=-1, keepdims=True)
        s = jnp.sum(z * active, axis=-1, keepdims=True)
        tau = (s - 1.0) / n
        active = jnp.where(mask & (z > tau), 1.0, 0.0)
    w = jnp.where(mask, jnp.maximum(z - tau, 0.0), 0.0)

    wrep = jnp.dot(w, rep_ref[...], preferred_element_type=jnp.float32)
    h = jnp.dot(xb, acat_ref[...], preferred_element_type=jnp.float32)
    eo = jnp.dot((wrep * h).astype(jnp.bfloat16), bcat_ref[...],
                 preferred_element_type=jnp.float32)

    base = jnp.dot(xb, wt_ref[...], preferred_element_type=jnp.float32)
    o_ref[...] = base + b_ref[...] + LORA_SCALING * eo


@functools.partial(jax.jit, static_argnames=())
def kernel(x, W_base, b_base, router_W, router_b, lora_A, lora_B):
    T = x.shape[0]
    Wt = W_base.T.astype(jnp.bfloat16)
    b2 = b_base.reshape(1, OUT_FEATURES)
    rw = jnp.zeros((IN_FEATURES, LANES), jnp.float32).at[:, :NUM_EXPERTS].set(
        router_W)
    rb = jnp.zeros((1, LANES), jnp.float32).at[0, :NUM_EXPERTS].set(router_b)
    acat = lora_A.transpose(1, 0, 2).reshape(
        IN_FEATURES, NUM_EXPERTS * LORA_RANK).astype(jnp.bfloat16)
    bcat = lora_B.reshape(NUM_EXPERTS * LORA_RANK,
                          OUT_FEATURES).astype(jnp.bfloat16)
    rep = np.zeros((LANES, LANES), np.float32)
    for e in range(NUM_EXPERTS):
        rep[e, e * LORA_RANK:(e + 1) * LORA_RANK] = 1.0
    rep = jnp.asarray(rep)

    grid = (T // BT,)
    out = pl.pallas_call(
        _body,
        grid=grid,
        in_specs=[
            pl.BlockSpec((BT, IN_FEATURES), lambda i: (i, 0)),
            pl.BlockSpec((IN_FEATURES, OUT_FEATURES), lambda i: (0, 0)),
            pl.BlockSpec((1, OUT_FEATURES), lambda i: (0, 0)),
            pl.BlockSpec((IN_FEATURES, LANES), lambda i: (0, 0)),
            pl.BlockSpec((1, LANES), lambda i: (0, 0)),
            pl.BlockSpec((IN_FEATURES, NUM_EXPERTS * LORA_RANK),
                         lambda i: (0, 0)),
            pl.BlockSpec((NUM_EXPERTS * LORA_RANK, OUT_FEATURES),
                         lambda i: (0, 0)),
            pl.BlockSpec((LANES, LANES), lambda i: (0, 0)),
        ],
        out_specs=pl.BlockSpec((BT, OUT_FEATURES), lambda i: (i, 0)),
        out_shape=jax.ShapeDtypeStruct((T, OUT_FEATURES), jnp.float32),
    )(x, Wt, b2, rw, rb, acat, bcat, rep)
    return out


# speedup vs baseline: 4.7900x; 1.0117x over previous
import functools

import jax
import jax.numpy as jnp
import numpy as np
from jax.experimental import pallas as pl

IN_FEATURES = 2048
OUT_FEATURES = 2048
NUM_EXPERTS = 8
LORA_RANK = 16
SPARSEGEN_LAMBDA = 0.5
LORA_SCALING = 2.0

LANES = 128
BT = 1024


def _body(x_ref, wt_ref, b_ref, rw_ref, rb_ref, acat_ref, bcat_ref, rep_ref,
          o_ref):
    x = x_ref[...]
    xb = x.astype(jnp.bfloat16)

    z = jnp.dot(x, rw_ref[...], preferred_element_type=jnp.float32)
    z = (z + rb_ref[...]) * (1.0 / (1.0 - SPARSEGEN_LAMBDA))
    lane = jax.lax.broadcasted_iota(jnp.int32, z.shape, dimension=1)
    mask = lane < NUM_EXPERTS
    active = mask.astype(jnp.float32)
    tau = jnp.zeros((z.shape[0], 1), jnp.float32)
    for _ in range(NUM_EXPERTS):
        n = jnp.sum(active, axis=-1, keepdims=True)
        s = jnp.sum(z * active, axis=-1, keepdims=True)
        tau = (s - 1.0) / n
        active = jnp.where(mask & (z > tau), 1.0, 0.0)
    w = jnp.where(mask, jnp.maximum(z - tau, 0.0), 0.0)

    wrep = jnp.dot(w, rep_ref[...], preferred_element_type=jnp.float32)
    h = jnp.dot(xb, acat_ref[...], preferred_element_type=jnp.float32)
    eo = jnp.dot((wrep * h).astype(jnp.bfloat16), bcat_ref[...],
                 preferred_element_type=jnp.float32)

    base = jnp.dot(xb, wt_ref[...], preferred_element_type=jnp.float32)
    o_ref[...] = base + b_ref[...] + LORA_SCALING * eo


@functools.partial(jax.jit, static_argnames=())
def kernel(x, W_base, b_base, router_W, router_b, lora_A, lora_B):
    T = x.shape[0]
    Wt = W_base.T.astype(jnp.bfloat16)
    b2 = b_base.reshape(1, OUT_FEATURES)
    rw = jnp.zeros((IN_FEATURES, LANES), jnp.float32).at[:, :NUM_EXPERTS].set(
        router_W)
    rb = jnp.zeros((1, LANES), jnp.float32).at[0, :NUM_EXPERTS].set(router_b)
    acat = lora_A.transpose(1, 0, 2).reshape(
        IN_FEATURES, NUM_EXPERTS * LORA_RANK).astype(jnp.bfloat16)
    bcat = lora_B.reshape(NUM_EXPERTS * LORA_RANK,
                          OUT_FEATURES).astype(jnp.bfloat16)
    rep = np.zeros((LANES, LANES), np.float32)
    for e in range(NUM_EXPERTS):
        rep[e, e * LORA_RANK:(e + 1) * LORA_RANK] = 1.0
    rep = jnp.asarray(rep)

    grid = (T // BT,)
    out = pl.pallas_call(
        _body,
        grid=grid,
        in_specs=[
            pl.BlockSpec((BT, IN_FEATURES), lambda i: (i, 0)),
            pl.BlockSpec((IN_FEATURES, OUT_FEATURES), lambda i: (0, 0)),
            pl.BlockSpec((1, OUT_FEATURES), lambda i: (0, 0)),
            pl.BlockSpec((IN_FEATURES, LANES), lambda i: (0, 0)),
            pl.BlockSpec((1, LANES), lambda i: (0, 0)),
            pl.BlockSpec((IN_FEATURES, NUM_EXPERTS * LORA_RANK),
                         lambda i: (0, 0)),
            pl.BlockSpec((NUM_EXPERTS * LORA_RANK, OUT_FEATURES),
                         lambda i: (0, 0)),
            pl.BlockSpec((LANES, LANES), lambda i: (0, 0)),
        ],
        out_specs=pl.BlockSpec((BT, OUT_FEATURES), lambda i: (i, 0)),
        out_shape=jax.ShapeDtypeStruct((T, OUT_FEATURES), jnp.float32),
    )(x, Wt, b2, rw, rb, acat, bcat, rep)
    return out
